# Initial kernel scaffold; baseline (speedup 1.0000x reference)
import jax
import jax.numpy as jnp
from jax.experimental import pallas as pl


def kernel(x, edge_index, edge_attr, batch, Wf1, bf1, Ws1, bs1, Wf2, bf2, Ws2, bs2, Wf3, bf3, Ws3, bs3, W_lin, b_lin):
    raise NotImplementedError("write your pallas kernel here")


# speedup vs baseline: 1.0136x; 1.0136x over previous
import functools

import jax
import jax.numpy as jnp
from jax.experimental import pallas as pl

E_BLK = 8000
N_GRAPHS_TOTAL = 64


def _msg_body(xd_ref, xs_ref, ea_ref, wfd_ref, wfs_ref, wfe_ref, bf_ref,
              wsd_ref, wss_ref, wse_ref, bs_ref, out_ref):
    xd = xd_ref[...]
    xs = xs_ref[...]
    ea = ea_ref[...]
    gf = (jnp.dot(xd, wfd_ref[...], preferred_element_type=jnp.float32)
          + jnp.dot(xs, wfs_ref[...], preferred_element_type=jnp.float32)
          + jnp.dot(ea, wfe_ref[...], preferred_element_type=jnp.float32)
          + bf_ref[...])
    gs = (jnp.dot(xd, wsd_ref[...], preferred_element_type=jnp.float32)
          + jnp.dot(xs, wss_ref[...], preferred_element_type=jnp.float32)
          + jnp.dot(ea, wse_ref[...], preferred_element_type=jnp.float32)
          + bs_ref[...])
    out_ref[...] = jax.nn.sigmoid(gf) * jax.nn.softplus(gs)


def _edge_messages(xd, xs, ea, Wf, bf, Ws, bs):
    n_edges, f_in = xd.shape
    d_edge = ea.shape[1]
    grid = (n_edges // E_BLK,)
    eb = lambda i: (i, 0)
    wb = lambda i: (0, 0)
    return pl.pallas_call(
        _msg_body,
        grid=grid,
        in_specs=[
            pl.BlockSpec((E_BLK, f_in), eb),
            pl.BlockSpec((E_BLK, f_in), eb),
            pl.BlockSpec((E_BLK, d_edge), eb),
            pl.BlockSpec((f_in, f_in), wb),
            pl.BlockSpec((f_in, f_in), wb),
            pl.BlockSpec((d_edge, f_in), wb),
            pl.BlockSpec((1, f_in), wb),
            pl.BlockSpec((f_in, f_in), wb),
            pl.BlockSpec((f_in, f_in), wb),
            pl.BlockSpec((d_edge, f_in), wb),
            pl.BlockSpec((1, f_in), wb),
        ],
        out_specs=pl.BlockSpec((E_BLK, f_in), eb),
        out_shape=jax.ShapeDtypeStruct((n_edges, f_in), jnp.float32),
    )(xd, xs, ea, Wf[:f_in], Wf[f_in:2 * f_in], Wf[2 * f_in:], bf[None, :],
      Ws[:f_in], Ws[f_in:2 * f_in], Ws[2 * f_in:], bs[None, :])


def _cgconv(x, src, dst, edge_attr, Wf, bf, Ws, bs):
    xd = x[dst]
    xs = x[src]
    msg = _edge_messages(xd, xs, edge_attr, Wf, bf, Ws, bs)
    agg = jax.ops.segment_sum(msg, dst, num_segments=x.shape[0])
    return x + agg


def _pool(x, batch, n_graphs):
    mx = jax.ops.segment_max(x, batch, num_segments=n_graphs)
    mx = jnp.where(jnp.isfinite(mx), mx, 0.0)
    s = jax.ops.segment_sum(x, batch, num_segments=n_graphs)
    cnt = jax.ops.segment_sum(jnp.ones((x.shape[0],), dtype=x.dtype), batch,
                              num_segments=n_graphs)
    mean = s / jnp.maximum(cnt, 1.0)[:, None]
    return jnp.concatenate([mx, mean], axis=1)


def kernel(x, edge_index, edge_attr, batch, Wf1, bf1, Ws1, bs1, Wf2, bf2,
           Ws2, bs2, Wf3, bf3, Ws3, bs3, W_lin, b_lin):
    n_graphs = N_GRAPHS_TOTAL
    src = edge_index[0]
    dst = edge_index[1]
    h = jax.nn.relu(_cgconv(x, src, dst, edge_attr, Wf1, bf1, Ws1, bs1))
    x1 = _pool(h, batch, n_graphs)
    h = jax.nn.relu(_cgconv(h, src, dst, edge_attr, Wf2, bf2, Ws2, bs2))
    x2 = _pool(h, batch, n_graphs)
    h = jax.nn.relu(_cgconv(h, src, dst, edge_attr, Wf3, bf3, Ws3, bs3))
    x3 = _pool(h, batch, n_graphs)
    enc = x1 + x2 + x3
    out = jax.nn.log_softmax(enc @ W_lin + b_lin, axis=-1)
    return (out, jax.lax.stop_gradient(enc))


# speedup vs baseline: 1.2143x; 1.1980x over previous
import functools

import jax
import jax.numpy as jnp
from jax import lax
from jax.experimental import pallas as pl
from jax.experimental.pallas import tpu as pltpu
from jax.experimental.pallas import tpu_sc as plsc

N_NODES = 100000
N_EDG = 1600000
F_IN = 50
ROW = 128
NPB = 1024
NB = (N_NODES + NPB - 1) // NPB
NBP = 112
NT = 32
ROUNDS = (NB + NT - 1) // NT
SLC = N_EDG // NT
CB = 80
CHK = 64
N_GRAPHS_TOTAL = 64
NBLK = 2000
EBLK = 8000

_C0 = 2.2117031201085435e-05
_C1 = 0.9990104466294503
_C2 = -0.489156847202284
_C3 = 0.2833043245174096
_C4 = -0.13011941539131197
_C5 = 0.03010262501171008

_mesh = plsc.VectorSubcoreMesh(core_axis_name="c", subcore_axis_name="s")
_SC_PARAMS = pltpu.CompilerParams(needs_layout_passes=False,
                                  use_tc_tiling_on_sc=False)


def _wid():
    return lax.axis_index("s") * 2 + lax.axis_index("c")


LW = 16 * NBP


@functools.partial(
    pl.kernel, mesh=_mesh, compiler_params=_SC_PARAMS,
    out_type=jax.ShapeDtypeStruct((NT, LW), jnp.int32),
    scratch_types=[
        pltpu.VMEM((SLC,), jnp.int32),
        pltpu.VMEM((LW,), jnp.int32),
        pltpu.SemaphoreType.DMA,
    ],
)
def _bin_count(dst_hbm, cnt_hbm, dvm, c16, sem):
    wid = _wid()
    pltpu.async_copy(dst_hbm.at[pl.ds(wid * SLC, SLC)], dvm, sem).wait()
    zi = jnp.zeros((16,), jnp.int32)

    def zero(i, _):
        c16[pl.ds(i * 16, 16)] = zi
        return 0

    lax.fori_loop(0, LW // 16, zero, 0)
    laneoff = lax.iota(jnp.int32, 16) * NBP
    ones = jnp.ones((16,), jnp.int32)

    def body(i, _):
        d = dvm[pl.ds(i * 16, 16)]
        idx = laneoff + (d >> 10)
        p = plsc.load_gather(c16, [idx])
        plsc.store_scatter(c16, [idx], p + ones)
        return 0

    lax.fori_loop(0, SLC // 16, body, 0)
    pltpu.sync_copy(c16, cnt_hbm.at[wid])


@functools.partial(
    pl.kernel, mesh=_mesh, compiler_params=_SC_PARAMS,
    out_type=(jax.ShapeDtypeStruct((NT, LW), jnp.int32),
              jax.ShapeDtypeStruct((NBP,), jnp.int32)),
    scratch_types=[
        pltpu.VMEM((NT, LW), jnp.int32),
        pltpu.VMEM((NT, LW), jnp.int32),
        pltpu.VMEM((NBP,), jnp.int32),
        pltpu.SemaphoreType.DMA,
    ],
)
def _bin_offsets(cnt_hbm, cur_hbm, off_hbm, cvm, uvm, ovm, sem):
    wid = _wid()

    @pl.when(wid == 0)
    def _():
        pltpu.async_copy(cnt_hbm, cvm, sem).wait()
        carry = jnp.int32(0)
        zi = jnp.zeros((16,), jnp.int32)
        for i in range(NBP // 16):
            def tl(t, v):
                def ll(l, v2):
                    return v2 + cvm[t, pl.ds(l * NBP + i * 16, 16)]

                return lax.fori_loop(0, 16, ll, v)

            tot = lax.fori_loop(0, NT, tl, zi)
            cs = plsc.cumsum(tot)
            ovm[pl.ds(i * 16, 16)] = cs - tot + carry
            carry = carry + jnp.sum(tot)
        for i in range(NBP // 16):
            start = ovm[pl.ds(i * 16, 16)]

            def tl2(t, cvec):
                def ll2(l, cv):
                    uvm[t, pl.ds(l * NBP + i * 16, 16)] = cv
                    return cv + cvm[t, pl.ds(l * NBP + i * 16, 16)]

                return lax.fori_loop(0, 16, ll2, cvec)

            lax.fori_loop(0, NT, tl2, start)
        pltpu.sync_copy(uvm, cur_hbm)
        pltpu.sync_copy(ovm, off_hbm)


@functools.partial(
    pl.kernel, mesh=_mesh, compiler_params=_SC_PARAMS,
    out_type=jax.ShapeDtypeStruct((N_EDG + CHK, 16), jnp.int32),
    scratch_types=[
        pltpu.VMEM((SLC,), jnp.int32),
        pltpu.VMEM((SLC,), jnp.int32),
        pltpu.VMEM((LW,), jnp.int32),
        pltpu.VMEM((CB,), jnp.int32),
        pltpu.VMEM((CB, 16), jnp.int32),
        pltpu.SemaphoreType.DMA,
        pltpu.SemaphoreType.DMA,
    ],
)
def _bin_scatter(dst_hbm, src_hbm, cur_hbm, pack_hbm,
                 dvm, svm, cur2, pos, pkl, sem, sem2):
    wid = _wid()
    base_e = wid * SLC
    c1 = pltpu.async_copy(dst_hbm.at[pl.ds(base_e, SLC)], dvm, sem)
    c2 = pltpu.async_copy(src_hbm.at[pl.ds(base_e, SLC)], svm, sem)
    c1.wait()
    c2.wait()
    pltpu.sync_copy(cur_hbm.at[wid], cur2)
    iota = lax.iota(jnp.int32, 16)
    laneoff = iota * NBP
    col0 = jnp.zeros((16,), jnp.int32)
    col1 = jnp.full((16,), 1, jnp.int32)
    col2 = jnp.full((16,), 2, jnp.int32)

    def chunk(ci, _):
        base = ci * CB
        for i in range(CB // 16):
            off = base + i * 16
            d = dvm[pl.ds(off, 16)]
            s = svm[pl.ds(off, 16)]
            idx = laneoff + (d >> 10)
            p = plsc.load_gather(cur2, [idx])
            plsc.store_scatter(cur2, [idx], p + 1)
            pos[pl.ds(i * 16, 16)] = p
            rows = i * 16 + iota
            plsc.store_scatter(pkl, [rows, col0], d)
            plsc.store_scatter(pkl, [rows, col1], s)
            plsc.store_scatter(pkl, [rows, col2], base_e + off + iota)
        pltpu.async_copy(pkl, pack_hbm.at[pos], sem2).wait()
        return 0

    lax.fori_loop(0, SLC // CB, chunk, 0)


def _poly_log1p(t):
    p = _C5 * t + _C4
    p = p * t + _C3
    p = p * t + _C2
    p = p * t + _C1
    return p * t + _C0


@functools.partial(
    pl.kernel, mesh=_mesh, compiler_params=_SC_PARAMS,
    out_type=jax.ShapeDtypeStruct((NB * NPB * F_IN,), jnp.float32),
    scratch_types=[
        pltpu.VMEM((NPB * F_IN + 64,), jnp.float32),
        pltpu.VMEM((128,), jnp.int32),
        pltpu.VMEM((CHK, 16), jnp.int32),
        pltpu.VMEM((CHK, 16), jnp.int32),
        pltpu.VMEM((CHK,), jnp.int32),
        pltpu.VMEM((CHK,), jnp.int32),
        pltpu.VMEM((CHK,), jnp.int32),
        pltpu.VMEM((CHK,), jnp.int32),
        pltpu.VMEM((CHK,), jnp.int32),
        pltpu.VMEM((CHK,), jnp.int32),
        pltpu.VMEM((CHK, ROW), jnp.float32),
        pltpu.VMEM((CHK, ROW), jnp.float32),
        pltpu.VMEM((CHK, ROW), jnp.float32),
        pltpu.VMEM((CHK, ROW), jnp.float32),
        pltpu.VMEM((CHK, ROW), jnp.float32),
        pltpu.VMEM((CHK, ROW), jnp.float32),
        pltpu.SemaphoreType.DMA,
        pltpu.SemaphoreType.DMA,
        pltpu.SemaphoreType.DMA,
        pltpu.SemaphoreType.DMA,
        pltpu.SemaphoreType.DMA,
    ],
)
def _edge_sc(pd_hbm, ps_hbm, e_hbm, pack_hbm, off_hbm, agg_hbm,
             acc, offs, pk0, pk1, dv0, sv0, ev0, dv1, sv1, ev1,
             rpd0, rps0, ree0, rpd1, rps1, ree1,
             psm0, psm1, rsm0, rsm1, osem):
    wid = _wid()
    pltpu.async_copy(off_hbm, offs.at[pl.ds(0, NBP)], osem).wait()
    iota = lax.iota(jnp.int32, 16)
    mask3 = (iota < 2).astype(jnp.float32)
    col0 = jnp.zeros((16,), jnp.int32)
    col1 = jnp.full((16,), 1, jnp.int32)
    col2 = jnp.full((16,), 2, jnp.int32)
    zf = jnp.zeros((16,), jnp.float32)

    pks = (pk0, pk1)
    dvs = (dv0, dv1)
    svs = (sv0, sv1)
    evs = (ev0, ev1)
    rpds = (rpd0, rpd1)
    rpss = (rps0, rps1)
    rees = (ree0, ree1)
    psms = (psm0, psm1)
    rsms = (rsm0, rsm1)

    def round_body(r, _):
        b = r * NT + wid

        @pl.when(b < NB)
        def _():
            def zero(i, _):
                acc[pl.ds(i * 16, 16)] = zf
                return 0

            lax.fori_loop(0, (NPB * F_IN + 64) // 16, zero, 0)
            ovec = offs[pl.ds(b, 16)]
            n0 = ovec[0]
            n1 = ovec[1]
            nch = (n1 - n0 + (CHK - 1)) >> 6
            base_node = b * NPB

            def fire_pack(c, h):
                pltpu.async_copy(pack_hbm.at[pl.ds(n0 + c * CHK, CHK)],
                                 pks[h], psms[h])

            def build_idx(h):
                pltpu.make_async_copy(
                    pack_hbm.at[pl.ds(0, CHK)], pks[h], psms[h]).wait()
                for j4 in range(4):
                    rows = j4 * 16 + iota
                    d = plsc.load_gather(pks[h], [rows, col0])
                    d = jnp.minimum(jnp.maximum(d, 0), N_NODES - 1)
                    dvs[h][pl.ds(j4 * 16, 16)] = d
                    s = plsc.load_gather(pks[h], [rows, col1])
                    s = jnp.minimum(jnp.maximum(s, 0), N_NODES - 1)
                    svs[h][pl.ds(j4 * 16, 16)] = s
                    e = plsc.load_gather(pks[h], [rows, col2])
                    e = jnp.minimum(jnp.maximum(e, 0), N_EDG - 1)
                    evs[h][pl.ds(j4 * 16, 16)] = e

            def fire_rows(h):
                pltpu.async_copy(pd_hbm.at[dvs[h]], rpds[h], rsms[h])
                pltpu.async_copy(ps_hbm.at[svs[h]], rpss[h], rsms[h])
                pltpu.async_copy(e_hbm.at[evs[h]], rees[h], rsms[h])

            def drain_rows(h):
                pltpu.make_async_copy(
                    pd_hbm.at[pl.ds(0, CHK)], rpds[h], rsms[h]).wait()
                pltpu.make_async_copy(
                    ps_hbm.at[pl.ds(0, CHK)], rpss[h], rsms[h]).wait()
                pltpu.make_async_copy(
                    e_hbm.at[pl.ds(0, CHK)], rees[h], rsms[h]).wait()

            def compute(c, h):
                e0c = n0 + c * CHK
                pk, pd, ps_, ee = pks[h], rpds[h], rpss[h], rees[h]

                def edge(j, _):
                    prow = pk[j, pl.ds(0, 16)]
                    d = prow[0]
                    dl = jnp.minimum(jnp.maximum(d - base_node, 0), NPB - 1)
                    addr = dl * F_IN
                    vf = jnp.where(e0c + j < n1, 1.0, 0.0).astype(jnp.float32)
                    for k in range(4):
                        gf = (pd[j, pl.ds(k * 16, 16)]
                              + ps_[j, pl.ds(k * 16, 16)]
                              + ee[j, pl.ds(k * 16, 16)])
                        gs = (pd[j, pl.ds(64 + k * 16, 16)]
                              + ps_[j, pl.ds(64 + k * 16, 16)]
                              + ee[j, pl.ds(64 + k * 16, 16)])
                        den = 1.0 + jnp.exp(-gf)
                        t = jnp.exp(-jnp.abs(gs))
                        sp = jnp.maximum(gs, 0.0) + _poly_log1p(t)
                        m = (sp / den) * vf
                        if k == 3:
                            m = m * mask3
                        plsc.addupdate(acc.at[pl.ds(addr + k * 16, 16)], m)
                    return 0

                lax.fori_loop(0, CHK, edge, 0)

            @pl.when(nch > 0)
            def _():
                fire_pack(0, 0)
                build_idx(0)
                fire_rows(0)

                @pl.when(nch > 1)
                def _():
                    fire_pack(1, 1)

            def sub(c, cur, nxt):
                @pl.when(c < nch)
                def _():
                    @pl.when(c + 1 < nch)
                    def _():
                        build_idx(nxt)
                        fire_rows(nxt)

                    drain_rows(cur)
                    compute(c, cur)

                    @pl.when(c + 2 < nch)
                    def _():
                        fire_pack(c + 2, cur)

            def gbody(g, _):
                sub(2 * g, 0, 1)
                sub(2 * g + 1, 1, 0)
                return 0

            lax.fori_loop(0, (nch + 1) >> 1, gbody, 0)
            pltpu.sync_copy(acc.at[pl.ds(0, NPB * F_IN)],
                            agg_hbm.at[pl.ds(b * NPB * F_IN, NPB * F_IN)])

        return 0

    lax.fori_loop(0, ROUNDS, round_body, 0)


def _proj_body(h_ref, wd_ref, ws_ref, pd_ref, ps_ref):
    h = h_ref[...]
    pd_ref[...] = jnp.dot(h, wd_ref[...], preferred_element_type=jnp.float32)
    ps_ref[...] = jnp.dot(h, ws_ref[...], preferred_element_type=jnp.float32)


def _proj(h, wd, ws):
    grid = (N_NODES // NBLK,)
    return pl.pallas_call(
        _proj_body,
        grid=grid,
        in_specs=[
            pl.BlockSpec((NBLK, F_IN), lambda i: (i, 0)),
            pl.BlockSpec((F_IN, ROW), lambda i: (0, 0)),
            pl.BlockSpec((F_IN, ROW), lambda i: (0, 0)),
        ],
        out_specs=[
            pl.BlockSpec((NBLK, ROW), lambda i: (i, 0)),
            pl.BlockSpec((NBLK, ROW), lambda i: (i, 0)),
        ],
        out_shape=[
            jax.ShapeDtypeStruct((N_NODES, ROW), jnp.float32),
            jax.ShapeDtypeStruct((N_NODES, ROW), jnp.float32),
        ],
    )(h, wd, ws)


def _eterm_body(ea_ref, we_ref, b_ref, out_ref):
    out_ref[...] = (jnp.dot(ea_ref[...], we_ref[...],
                            preferred_element_type=jnp.float32) + b_ref[...])


def _eterm(ea, we, ball):
    d_edge = ea.shape[1]
    grid = (N_EDG // EBLK,)
    return pl.pallas_call(
        _eterm_body,
        grid=grid,
        in_specs=[
            pl.BlockSpec((EBLK, d_edge), lambda i: (i, 0)),
            pl.BlockSpec((d_edge, ROW), lambda i: (0, 0)),
            pl.BlockSpec((1, ROW), lambda i: (0, 0)),
        ],
        out_specs=pl.BlockSpec((EBLK, ROW), lambda i: (i, 0)),
        out_shape=jax.ShapeDtypeStruct((N_EDG, ROW), jnp.float32),
    )(ea, we, ball)


def _update_pool_body(h_ref, agg_ref, bf_ref, hout_ref, sum_ref, mx_ref,
                      cnt_ref):
    h2 = jax.nn.relu(h_ref[...] + agg_ref[...])
    hout_ref[...] = h2
    bcol = bf_ref[...]
    gids = lax.broadcasted_iota(
        jnp.int32, (1, N_GRAPHS_TOTAL), 1).astype(jnp.float32)
    ohb = bcol == gids
    oh = ohb.astype(jnp.float32)
    psum = lax.dot_general(oh, h2, (((0,), (0,)), ((), ())),
                           preferred_element_type=jnp.float32)
    pcnt = jnp.sum(oh, axis=0)[None, :]
    parts = []
    for g in range(N_GRAPHS_TOTAL):
        mcol = oh[:, g:g + 1]
        masked = h2 + (mcol - 1.0) * 1e30
        parts.append(jnp.max(masked, axis=0, keepdims=True))
    pmax = jnp.concatenate(parts, axis=0)

    @pl.when(pl.program_id(0) == 0)
    def _():
        sum_ref[...] = jnp.zeros_like(sum_ref)
        mx_ref[...] = jnp.full_like(mx_ref, -1e30)
        cnt_ref[...] = jnp.zeros_like(cnt_ref)

    sum_ref[...] += psum
    cnt_ref[...] += pcnt
    mx_ref[...] = jnp.maximum(mx_ref[...], pmax)


def _update_pool(h, agg, batchf):
    grid = (N_NODES // NBLK,)
    return pl.pallas_call(
        _update_pool_body,
        grid=grid,
        in_specs=[
            pl.BlockSpec((NBLK, F_IN), lambda i: (i, 0)),
            pl.BlockSpec((NBLK, F_IN), lambda i: (i, 0)),
            pl.BlockSpec((NBLK, 1), lambda i: (i, 0)),
        ],
        out_specs=[
            pl.BlockSpec((NBLK, F_IN), lambda i: (i, 0)),
            pl.BlockSpec((N_GRAPHS_TOTAL, F_IN), lambda i: (0, 0)),
            pl.BlockSpec((N_GRAPHS_TOTAL, F_IN), lambda i: (0, 0)),
            pl.BlockSpec((1, N_GRAPHS_TOTAL), lambda i: (0, 0)),
        ],
        out_shape=[
            jax.ShapeDtypeStruct((N_NODES, F_IN), jnp.float32),
            jax.ShapeDtypeStruct((N_GRAPHS_TOTAL, F_IN), jnp.float32),
            jax.ShapeDtypeStruct((N_GRAPHS_TOTAL, F_IN), jnp.float32),
            jax.ShapeDtypeStruct((1, N_GRAPHS_TOTAL), jnp.float32),
        ],
    )(h, agg, batchf)


def _head_body(s1_ref, s2_ref, s3_ref, m1_ref, m2_ref, m3_ref, cnt_ref,
               wl_ref, bl_ref, out_ref, enc_ref):
    cnt = jnp.maximum(jnp.reshape(cnt_ref[...], (N_GRAPHS_TOTAL, 1)), 1.0)
    enc = jnp.zeros((N_GRAPHS_TOTAL, 2 * F_IN), jnp.float32)
    for s_ref, m_ref in ((s1_ref, m1_ref), (s2_ref, m2_ref), (s3_ref, m3_ref)):
        mx = jnp.maximum(m_ref[...], 0.0)
        mean = s_ref[...] / cnt
        enc = enc + jnp.concatenate([mx, mean], axis=1)
    enc_ref[...] = enc
    logits = jnp.dot(enc, wl_ref[...],
                     preferred_element_type=jnp.float32) + bl_ref[...]
    mxl = jnp.max(logits, axis=1, keepdims=True)
    lse = mxl + jnp.log(jnp.sum(jnp.exp(logits - mxl), axis=1, keepdims=True))
    out_ref[...] = logits - lse


def _head(s1, s2, s3, m1, m2, m3, cnt, wl, bl):
    n_out = wl.shape[1]
    full = lambda shape: pl.BlockSpec(shape, lambda: tuple(0 for _ in shape))
    return pl.pallas_call(
        _head_body,
        in_specs=[
            full((N_GRAPHS_TOTAL, F_IN))] * 6 + [
            full((1, N_GRAPHS_TOTAL)),
            full((2 * F_IN, n_out)),
            full((1, n_out)),
        ],
        out_specs=[
            full((N_GRAPHS_TOTAL, n_out)),
            full((N_GRAPHS_TOTAL, 2 * F_IN)),
        ],
        out_shape=[
            jax.ShapeDtypeStruct((N_GRAPHS_TOTAL, n_out), jnp.float32),
            jax.ShapeDtypeStruct((N_GRAPHS_TOTAL, 2 * F_IN), jnp.float32),
        ],
    )(s1, s2, s3, m1, m2, m3, cnt, wl, bl)


def _pack_w(wf, ws):
    pad = lambda w: jnp.pad(w, ((0, 0), (0, 64 - F_IN)))
    return jnp.concatenate([pad(wf), pad(ws)], axis=1)


def kernel(x, edge_index, edge_attr, batch, Wf1, bf1, Ws1, bs1, Wf2, bf2,
           Ws2, bs2, Wf3, bf3, Ws3, bs3, W_lin, b_lin):
    src = edge_index[0]
    dst = edge_index[1]
    batchf = batch.astype(jnp.float32)[:, None]

    cnts = _bin_count(dst)
    curs, offs = _bin_offsets(cnts)
    pack = _bin_scatter(dst, src, curs)

    layers = ((Wf1, bf1, Ws1, bs1), (Wf2, bf2, Ws2, bs2), (Wf3, bf3, Ws3, bs3))
    h = x
    sums, maxs, cnt = [], [], None
    for Wf, bf, Ws, bs in layers:
        wd = _pack_w(Wf[:F_IN], Ws[:F_IN])
        ws_ = _pack_w(Wf[F_IN:2 * F_IN], Ws[F_IN:2 * F_IN])
        we = _pack_w(Wf[2 * F_IN:], Ws[2 * F_IN:])
        ball = _pack_w(bf[None, :], bs[None, :])
        pd, ps_ = _proj(h, wd, ws_)
        ee = _eterm(edge_attr, we, ball)
        agg_flat = _edge_sc(pd, ps_, ee, pack, offs)
        agg = jnp.reshape(agg_flat, (NB * NPB, F_IN))[:N_NODES]
        h, psum, pmax, pcnt = _update_pool(h, agg, batchf)
        sums.append(psum)
        maxs.append(pmax)
        if cnt is None:
            cnt = pcnt
    out, enc = _head(sums[0], sums[1], sums[2], maxs[0], maxs[1], maxs[2],
                     cnt, W_lin, b_lin[None, :])
    return (out, jax.lax.stop_gradient(enc))
